# baseline (speedup 1.0000x reference)
import functools

import jax
import jax.numpy as jnp
from jax import lax
from jax.experimental import pallas as pl
from jax.experimental.pallas import tpu as pltpu
from jax.experimental.pallas import tpu_sc as plsc

N, E, D = 10000, 320000, 128
NC, NS, L = 2, 16, 16
NW = NC * NS
K = 64
NCHUNK_ALL = E // K
NCHUNK_LO = NCHUNK_ALL // NW
NREM = NCHUNK_ALL - NCHUNK_LO * NW
NGROUP = K // L
RPT = 624

_mesh = plsc.VectorSubcoreMesh(core_axis_name="c", subcore_axis_name="s")

_BCAST_DNUMS = lax.GatherDimensionNumbers(
    offset_dims=(), collapsed_slice_dims=(0,), start_index_map=(0,))


def _bcast_lane(vec, l):
  return lax.gather(
      vec,
      jnp.full((L, 1), l, dtype=jnp.int32),
      _BCAST_DNUMS,
      slice_sizes=(1,),
      mode=lax.GatherScatterMode.PROMISE_IN_BOUNDS)


@functools.partial(
    pl.kernel,
    mesh=_mesh,
    out_type=[
        jax.ShapeDtypeStruct((E, D), jnp.float32),
        jax.ShapeDtypeStruct((E, D), jnp.float32),
    ],
    scratch_types=[
        pltpu.VMEM((K,), jnp.int32),
        pltpu.VMEM((K,), jnp.int32),
        pltpu.VMEM((K,), jnp.int32),
        pltpu.VMEM((K,), jnp.int32),
        pltpu.VMEM((K,), jnp.int32),
        pltpu.VMEM((K,), jnp.int32),
        pltpu.VMEM((K,), jnp.float32),
        pltpu.VMEM((K,), jnp.float32),
        pltpu.VMEM((K,), jnp.float32),
        pltpu.VMEM((K, D), jnp.float32),
        pltpu.VMEM((K, D), jnp.float32),
        pltpu.VMEM((K, D), jnp.float32),
        pltpu.VMEM((K, D), jnp.float32),
        pltpu.VMEM((K, D), jnp.float32),
        pltpu.VMEM((K, D), jnp.float32),
        pltpu.VMEM_SHARED((N, D), jnp.float32),
        pltpu.SemaphoreType.DMA,
        pltpu.SemaphoreType.DMA,
        pltpu.SemaphoreType.DMA,
        pltpu.SemaphoreType.DMA,
        pltpu.SemaphoreType.DMA,
        pltpu.SemaphoreType.DMA,
        pltpu.SemaphoreType.DMA,
        pltpu.SemaphoreType.DMA,
        pltpu.SemaphoreType.DMA,
    ],
)
def _n2e(xn_hbm, src_hbm, dst_hbm, w_hbm, grad_hbm, ave_hbm,
         si0, si1, si2, di0, di1, di2, wv0, wv1, wv2,
         sr0, sr1, sr2, dr0, dr1, dr2,
         xnsh, g0, g1, g2, i0, i1, i2, o0, o1, o2):
  si = (si0, si1, si2)
  di = (di0, di1, di2)
  wv = (wv0, wv1, wv2)
  sr = (sr0, sr1, sr2)
  dr = (dr0, dr1, dr2)
  gsem = (g0, g1, g2)
  isem = (i0, i1, i2)
  osem = (o0, o1, o2)
  wid = lax.axis_index("s") * NC + lax.axis_index("c")
  nchunk = NCHUNK_LO + jnp.where(wid < NREM, 1, 0)

  def cbase(c):
    return (wid + NW * c) * K

  def drain_out(b):
    pltpu.make_async_copy(sr[b], grad_hbm.at[pl.ds(0, K)], osem[b]).wait()
    pltpu.make_async_copy(dr[b], ave_hbm.at[pl.ds(0, K)], osem[b]).wait()

  def drain_idx(b):
    pltpu.make_async_copy(src_hbm.at[pl.ds(0, K)], si[b], isem[b]).wait()
    pltpu.make_async_copy(dst_hbm.at[pl.ds(0, K)], di[b], isem[b]).wait()
    pltpu.make_async_copy(w_hbm.at[pl.ds(0, K)], wv[b], isem[b]).wait()

  def drain_gather(b):
    pltpu.make_async_copy(xnsh.at[si[b]], sr[b], gsem[b]).wait()
    pltpu.make_async_copy(xnsh.at[di[b]], dr[b], gsem[b]).wait()

  def issue_idx(c, b):
    base = cbase(c)
    pltpu.async_copy(src_hbm.at[pl.ds(base, K)], si[b], isem[b])
    pltpu.async_copy(dst_hbm.at[pl.ds(base, K)], di[b], isem[b])

  def issue_w(c, b):
    base = cbase(c)
    pltpu.async_copy(w_hbm.at[pl.ds(base, K)], wv[b], isem[b])

  def issue_gather(b):
    pltpu.async_copy(xnsh.at[si[b]], sr[b], gsem[b])
    pltpu.async_copy(xnsh.at[di[b]], dr[b], gsem[b])

  def issue_out(c, b):
    base = cbase(c)
    pltpu.async_copy(sr[b], grad_hbm.at[pl.ds(base, K)], osem[b])
    pltpu.async_copy(dr[b], ave_hbm.at[pl.ds(base, K)], osem[b])

  def compute(b):
    def group_body(g, c2):
      wg = wv[b][pl.ds(g * L, L)]
      for l in range(L):
        e = g * L + l
        wfull = _bcast_lane(wg, l)
        whalf = wfull * 0.5
        for j in range(D // L):
          sl = pl.ds(j * L, L)
          s = sr[b][e, sl]
          d = dr[b][e, sl]
          sr[b][e, sl] = wfull * (s - d)
          dr[b][e, sl] = whalf * (s + d)
      return c2

    lax.fori_loop(0, NGROUP, group_body, 0, unroll=2)

  def section(c, b):
    bp1 = (b + 1) % 3
    bp2 = (b + 2) % 3

    @pl.when(c < nchunk)
    def _():
      @pl.when(c >= 2)
      def _():
        drain_out(bp1)

      @pl.when(c + 1 < nchunk)
      def _():
        drain_idx(bp1)
        issue_gather(bp1)

      drain_gather(b)

      compute(b)

      issue_out(c, b)

      @pl.when(c + 3 < nchunk)
      def _():
        issue_idx(c + 3, b)
        issue_w(c + 3, b)

  sid = lax.axis_index("s")

  @pl.when(sid < NS - 1)
  def _():
    pltpu.sync_copy(xn_hbm.at[pl.ds(sid * RPT, RPT)],
                    xnsh.at[pl.ds(sid * RPT, RPT)])

  @pl.when(sid == NS - 1)
  def _():
    pltpu.sync_copy(xn_hbm.at[pl.ds((NS - 1) * RPT, N - (NS - 1) * RPT)],
                    xnsh.at[pl.ds((NS - 1) * RPT, N - (NS - 1) * RPT)])

  plsc.subcore_barrier()

  base0 = cbase(0)
  pltpu.sync_copy(src_hbm.at[pl.ds(base0, K)], si[0])
  pltpu.sync_copy(dst_hbm.at[pl.ds(base0, K)], di[0])
  pltpu.sync_copy(w_hbm.at[pl.ds(base0, K)], wv[0])
  issue_gather(0)
  issue_idx(1, 1)
  issue_w(1, 1)
  issue_idx(2, 2)
  issue_w(2, 2)

  def outer_body(io, carry):
    section(3 * io, 0)
    section(3 * io + 1, 1)
    section(3 * io + 2, 2)
    return carry

  lax.fori_loop(0, (NCHUNK_LO + 1 + 2) // 3, outer_body, 0, unroll=False)

  @pl.when(wid < NREM)
  def _():
    drain_out(2)
    drain_out(0)

  @pl.when(wid >= NREM)
  def _():
    drain_out(1)
    drain_out(2)


def kernel(xn, xe_src, xe_dst, W):
  src = xe_src.astype(jnp.int32)
  dst = xe_dst.astype(jnp.int32)
  w = W.reshape(-1).astype(jnp.float32)
  grad, ave = _n2e(xn, src, dst, w)
  return grad, ave


# speedup vs baseline: 1.1927x; 1.1927x over previous
import functools

import jax
import jax.numpy as jnp
from jax import lax
from jax.experimental import pallas as pl
from jax.experimental.pallas import tpu as pltpu
from jax.experimental.pallas import tpu_sc as plsc

N, E, D = 10000, 320000, 128
NC, NS, L = 2, 16, 16
NW = NC * NS
K = 64
NCHUNK_ALL = E // K
NCHUNK_LO = NCHUNK_ALL // NW
NREM = NCHUNK_ALL - NCHUNK_LO * NW
NGROUP = K // L
RPT = 624

_mesh = plsc.VectorSubcoreMesh(core_axis_name="c", subcore_axis_name="s")

_BCAST_DNUMS = lax.GatherDimensionNumbers(
    offset_dims=(), collapsed_slice_dims=(0,), start_index_map=(0,))


def _bcast_lane(vec, l):
  return lax.gather(
      vec,
      jnp.full((L, 1), l, dtype=jnp.int32),
      _BCAST_DNUMS,
      slice_sizes=(1,),
      mode=lax.GatherScatterMode.PROMISE_IN_BOUNDS)


@functools.partial(
    pl.kernel,
    mesh=_mesh,
    out_type=[
        jax.ShapeDtypeStruct((E, D), jnp.float32),
        jax.ShapeDtypeStruct((E, D), jnp.float32),
    ],
    scratch_types=[
        pltpu.VMEM((K,), jnp.int32),
        pltpu.VMEM((K,), jnp.int32),
        pltpu.VMEM((K,), jnp.int32),
        pltpu.VMEM((K,), jnp.int32),
        pltpu.VMEM((K,), jnp.int32),
        pltpu.VMEM((K,), jnp.int32),
        pltpu.VMEM((K,), jnp.float32),
        pltpu.VMEM((K,), jnp.float32),
        pltpu.VMEM((K,), jnp.float32),
        pltpu.VMEM((K, D), jnp.float32),
        pltpu.VMEM((K, D), jnp.float32),
        pltpu.VMEM((K, D), jnp.float32),
        pltpu.VMEM((K, D), jnp.float32),
        pltpu.VMEM((K, D), jnp.float32),
        pltpu.VMEM((K, D), jnp.float32),
        pltpu.VMEM_SHARED((N, D), jnp.float32),
        pltpu.SemaphoreType.DMA,
        pltpu.SemaphoreType.DMA,
        pltpu.SemaphoreType.DMA,
        pltpu.SemaphoreType.DMA,
        pltpu.SemaphoreType.DMA,
        pltpu.SemaphoreType.DMA,
        pltpu.SemaphoreType.DMA,
        pltpu.SemaphoreType.DMA,
        pltpu.SemaphoreType.DMA,
    ],
)
def _n2e(xn_hbm, src_hbm, dst_hbm, w_hbm, grad_hbm, ave_hbm,
         si0, si1, si2, di0, di1, di2, wv0, wv1, wv2,
         sr0, sr1, sr2, dr0, dr1, dr2,
         xnsh, g0, g1, g2, i0, i1, i2, o0, o1, o2):
  si = (si0, si1, si2)
  di = (di0, di1, di2)
  wv = (wv0, wv1, wv2)
  sr = (sr0, sr1, sr2)
  dr = (dr0, dr1, dr2)
  gsem = (g0, g1, g2)
  isem = (i0, i1, i2)
  osem = (o0, o1, o2)
  wid = lax.axis_index("s") * NC + lax.axis_index("c")
  nchunk = NCHUNK_LO + jnp.where(wid < NREM, 1, 0)

  def cbase(c):
    return (wid + NW * c) * K

  def drain_out(b):
    pltpu.make_async_copy(sr[b], grad_hbm.at[pl.ds(0, K)], osem[b]).wait()
    pltpu.make_async_copy(dr[b], ave_hbm.at[pl.ds(0, K)], osem[b]).wait()

  def drain_idx(b):
    pltpu.make_async_copy(src_hbm.at[pl.ds(0, K)], si[b], isem[b]).wait()
    pltpu.make_async_copy(dst_hbm.at[pl.ds(0, K)], di[b], isem[b]).wait()
    pltpu.make_async_copy(w_hbm.at[pl.ds(0, K)], wv[b], isem[b]).wait()

  def drain_gather(b):
    pltpu.make_async_copy(xnsh.at[si[b]], sr[b], gsem[b]).wait()
    pltpu.make_async_copy(xnsh.at[di[b]], dr[b], gsem[b]).wait()

  def issue_idx(c, b):
    base = cbase(c)
    pltpu.async_copy(src_hbm.at[pl.ds(base, K)], si[b], isem[b])
    pltpu.async_copy(dst_hbm.at[pl.ds(base, K)], di[b], isem[b])

  def issue_w(c, b):
    base = cbase(c)
    pltpu.async_copy(w_hbm.at[pl.ds(base, K)], wv[b], isem[b])

  def issue_gather(b):
    pltpu.async_copy(xnsh.at[si[b]], sr[b], gsem[b])
    pltpu.async_copy(xnsh.at[di[b]], dr[b], gsem[b])

  def issue_out(c, b):
    base = cbase(c)
    pltpu.async_copy(sr[b], grad_hbm.at[pl.ds(base, K)], osem[b])
    pltpu.async_copy(dr[b], ave_hbm.at[pl.ds(base, K)], osem[b])

  def compute(b):
    def group_body(g, c2):
      wg = wv[b][pl.ds(g * L, L)]
      for l in range(L):
        e = g * L + l
        wfull = _bcast_lane(wg, l)
        whalf = wfull * 0.5
        for j in range(D // L):
          sl = pl.ds(j * L, L)
          s = sr[b][e, sl]
          d = dr[b][e, sl]
          sr[b][e, sl] = wfull * (s - d)
          dr[b][e, sl] = whalf * (s + d)
      return c2

    lax.fori_loop(0, NGROUP, group_body, 0, unroll=False)

  def section(c, b):
    bp1 = (b + 1) % 3
    bp2 = (b + 2) % 3

    @pl.when(c < nchunk)
    def _():
      @pl.when(c >= 2)
      def _():
        drain_out(bp1)

      @pl.when(c + 1 < nchunk)
      def _():
        drain_idx(bp1)
        issue_gather(bp1)

      drain_gather(b)

      compute(b)

      issue_out(c, b)

      @pl.when(c + 3 < nchunk)
      def _():
        issue_idx(c + 3, b)
        issue_w(c + 3, b)

  sid = lax.axis_index("s")

  @pl.when(sid < NS - 1)
  def _():
    pltpu.sync_copy(xn_hbm.at[pl.ds(sid * RPT, RPT)],
                    xnsh.at[pl.ds(sid * RPT, RPT)])

  @pl.when(sid == NS - 1)
  def _():
    pltpu.sync_copy(xn_hbm.at[pl.ds((NS - 1) * RPT, N - (NS - 1) * RPT)],
                    xnsh.at[pl.ds((NS - 1) * RPT, N - (NS - 1) * RPT)])

  plsc.subcore_barrier()

  base0 = cbase(0)
  pltpu.sync_copy(src_hbm.at[pl.ds(base0, K)], si[0])
  pltpu.sync_copy(dst_hbm.at[pl.ds(base0, K)], di[0])
  pltpu.sync_copy(w_hbm.at[pl.ds(base0, K)], wv[0])
  issue_gather(0)
  issue_idx(1, 1)
  issue_w(1, 1)
  issue_idx(2, 2)
  issue_w(2, 2)

  def outer_body(io, carry):
    section(3 * io, 0)
    section(3 * io + 1, 1)
    section(3 * io + 2, 2)
    return carry

  lax.fori_loop(0, (NCHUNK_LO + 1 + 2) // 3, outer_body, 0, unroll=False)

  @pl.when(wid < NREM)
  def _():
    drain_out(2)
    drain_out(0)

  @pl.when(wid >= NREM)
  def _():
    drain_out(1)
    drain_out(2)


def kernel(xn, xe_src, xe_dst, W):
  src = xe_src.astype(jnp.int32)
  dst = xe_dst.astype(jnp.int32)
  w = W.reshape(-1).astype(jnp.float32)
  grad, ave = _n2e(xn, src, dst, w)
  return grad, ave
